# Initial kernel scaffold; baseline (speedup 1.0000x reference)
import jax
import jax.numpy as jnp
from jax.experimental import pallas as pl


def kernel(x, edge_index0, edge_index1, edge_index2, Wl0, Wr0, b0, Wl1, Wr1, b1, Wl2, Wr2, b2):
    raise NotImplementedError("write your pallas kernel here")


# speedup vs baseline: 3.1226x; 3.1226x over previous
import functools

import jax
import jax.numpy as jnp
from jax import lax
from jax.experimental import pallas as pl
from jax.experimental.pallas import tpu as pltpu
from jax.experimental.pallas import tpu_sc as plsc

N = 10000
D = 128
E = 320000
NC = 2
NS = 16
NW = NC * NS
CHUNK = 128
EPW = -(-E // (NW * CHUNK)) * CHUNK
EPAD = EPW * NW
NCHUNKS = EPW // CHUNK
NPAD = ((N + NS * CHUNK - 1) // (NS * CHUNK)) * NS * CHUNK
RPT = NPAD // NS
RCH = RPT // CHUNK


def _sc_counts_body(dst0_hbm, dst1_hbm, dst2_hbm, ones_hbm, zrow_hbm,
                    c0_hbm, c1_hbm, c2_hbm, dst_idx, ones128, acc):
    cid = lax.axis_index("c")
    sid = lax.axis_index("s")
    pltpu.sync_copy(ones_hbm, ones128)
    r0 = sid * RPT
    base0 = (cid * NS + sid) * EPW
    for dst_hbm, c_hbm in ((dst0_hbm, c0_hbm), (dst1_hbm, c1_hbm),
                           (dst2_hbm, c2_hbm)):
        def zero(i, _):
            pltpu.sync_copy(zrow_hbm, acc.at[pl.ds(r0 + i * CHUNK, CHUNK)])
            return 0
        lax.fori_loop(0, RCH, zero, 0, unroll=False)
        plsc.subcore_barrier()

        def chunk(g, _):
            pltpu.sync_copy(dst_hbm.at[pl.ds(base0 + g * CHUNK, CHUNK)], dst_idx)
            pltpu.sync_copy(ones128, acc.at[dst_idx], add=True)
            return 0
        lax.fori_loop(0, NCHUNKS, chunk, 0, unroll=False)
        plsc.subcore_barrier()

        def out(i, _):
            r = r0 + i * CHUNK
            pltpu.sync_copy(acc.at[pl.ds(r, CHUNK)], c_hbm.at[cid, pl.ds(r, CHUNK)])
            return 0
        lax.fori_loop(0, RCH, out, 0, unroll=False)
        plsc.subcore_barrier()


def _sc_agg_body(x_hbm, src_hbm, dst_hbm, zrow_hbm, sums_hbm,
                 src_idx, dst_idx, rows, acc, sem):
    cid = lax.axis_index("c")
    sid = lax.axis_index("s")
    r0 = sid * RPT

    def zero(i, _):
        pltpu.sync_copy(zrow_hbm, acc.at[pl.ds(r0 + i * CHUNK, CHUNK)])
        return 0

    lax.fori_loop(0, RCH, zero, 0, unroll=False)
    plsc.subcore_barrier()

    base0 = (cid * NS + sid) * EPW

    def chunk(g, _):
        base = base0 + g * CHUNK
        pltpu.sync_copy(src_hbm.at[pl.ds(base, CHUNK)], src_idx)
        pltpu.sync_copy(dst_hbm.at[pl.ds(base, CHUNK)], dst_idx)
        pltpu.async_copy(x_hbm.at[src_idx], rows, sem).wait()
        pltpu.sync_copy(rows, acc.at[dst_idx], add=True)
        return 0

    lax.fori_loop(0, NCHUNKS, chunk, 0, unroll=False)
    plsc.subcore_barrier()

    def out(i, _):
        r = r0 + i * CHUNK
        pltpu.sync_copy(acc.at[pl.ds(r, CHUNK)], sums_hbm.at[cid, pl.ds(r, CHUNK)])
        return 0

    lax.fori_loop(0, RCH, out, 0, unroll=False)


_SC_MESH = plsc.VectorSubcoreMesh(core_axis_name="c", subcore_axis_name="s",
                                  num_cores=NC, num_subcores=NS)

_COUNT_OUT = [jax.ShapeDtypeStruct((NC, NPAD, D), jnp.float32)] * 3
_COUNT_SCRATCH = [
    pltpu.VMEM((CHUNK,), jnp.int32),
    pltpu.VMEM((CHUNK, D), jnp.float32),
    pltpu.VMEM_SHARED((NPAD, D), jnp.float32),
]

_sc_counts = pl.kernel(
    _sc_counts_body,
    out_type=_COUNT_OUT,
    mesh=_SC_MESH,
    scratch_types=_COUNT_SCRATCH,
)

_AGG_OUT = jax.ShapeDtypeStruct((NC, NPAD, D), jnp.float32)
_AGG_SCRATCH = [
    pltpu.VMEM((CHUNK,), jnp.int32),
    pltpu.VMEM((CHUNK,), jnp.int32),
    pltpu.VMEM((CHUNK, D), jnp.float32),
    pltpu.VMEM_SHARED((NPAD, D), jnp.float32),
    pltpu.SemaphoreType.DMA,
]

_sc_aggregate = pl.kernel(
    _sc_agg_body,
    out_type=_AGG_OUT,
    mesh=_SC_MESH,
    scratch_types=_AGG_SCRATCH,
)


BR = 1000


def _tc_body(sums_ref, cnts_ref, x_ref, wl_ref, wr_ref, b_ref, o_ref, *, last):
    s = sums_ref[0] + sums_ref[1]
    c = cnts_ref[0, :, 0] + cnts_ref[1, :, 0]
    mean = s / jnp.maximum(c, 1.0)[:, None]
    out = jnp.dot(mean, wl_ref[...], preferred_element_type=jnp.float32)
    out = out + jnp.dot(x_ref[...], wr_ref[...], preferred_element_type=jnp.float32)
    out = out + b_ref[...]
    if not last:
        out = jax.nn.gelu(out)
    o_ref[...] = out


def _tc_combine(sums, cnts, x, wl, wr, b, last):
    return pl.pallas_call(
        functools.partial(_tc_body, last=last),
        grid=(N // BR,),
        in_specs=[
            pl.BlockSpec((NC, BR, D), lambda i: (0, i, 0)),
            pl.BlockSpec((NC, BR, D), lambda i: (0, i, 0)),
            pl.BlockSpec((BR, D), lambda i: (i, 0)),
            pl.BlockSpec((D, D), lambda i: (0, 0)),
            pl.BlockSpec((D, D), lambda i: (0, 0)),
            pl.BlockSpec((1, D), lambda i: (0, 0)),
        ],
        out_specs=pl.BlockSpec((BR, D), lambda i: (i, 0)),
        out_shape=jax.ShapeDtypeStruct((N, D), jnp.float32),
    )(sums, cnts, x, wl, wr, b)


def kernel(x, edge_index0, edge_index1, edge_index2,
           Wl0, Wr0, b0, Wl1, Wr1, b1, Wl2, Wr2, b2):
    eis = (edge_index0, edge_index1, edge_index2)
    params = ((Wl0, Wr0, b0), (Wl1, Wr1, b1), (Wl2, Wr2, b2))
    pad = EPAD - E
    srcs = [jnp.concatenate([ei[0], jnp.zeros((pad,), jnp.int32)]) for ei in eis]
    dsts = [jnp.concatenate([ei[1], jnp.full((pad,), N, jnp.int32)]) for ei in eis]
    ones128 = jnp.ones((CHUNK, D), jnp.float32)
    zrow = jnp.zeros((CHUNK, D), jnp.float32)
    cnts = _sc_counts(dsts[0], dsts[1], dsts[2], ones128, zrow)
    for i in range(3):
        sums = _sc_aggregate(x, srcs[i], dsts[i], zrow)
        wl, wr, b = params[i]
        x = _tc_combine(sums, cnts[i], x, wl, wr, b.reshape(1, D), last=(i == 2))
    return x
